# baseline (speedup 1.0000x reference)
import jax
import jax.numpy as jnp
import numpy as np
from jax.experimental import pallas as pl
from jax.experimental.pallas import tpu as pltpu

SPW = 16
IMG_W = 224
GRID = IMG_W // SPW
CH = 3
BBLK = 4
RING = 8
FR = 392
NT = 7


def _mean_body(img_ref, out_ref):
    out_ref[0, 0] = jnp.sum(img_ref[...]) * (1.0 / (CH * IMG_W * IMG_W))


def _pix_body(xg_ref, imgf_ref, psc_ref, sel_ref, mean_ref, out_ref, buf, sem):
    i = pl.program_id(0)
    nb = pl.num_programs(0) * BBLK
    m = mean_ref[0, 0]
    imgf = imgf_ref[...]
    for j in range(BBLK):
        b = i * BBLK + j
        slot = b % RING

        @pl.when(b >= RING)
        def _wait_prev():
            pltpu.make_async_copy(buf.at[slot], out_ref.at[b - RING],
                                  sem.at[slot]).wait()

        g = xg_ref[j]
        q = jnp.zeros((FR, 128), jnp.float32)
        for t in range(NT):
            bt = jnp.dot(g, sel_ref[t], preferred_element_type=jnp.float32)
            q = q + jnp.dot(psc_ref[t], bt, preferred_element_type=jnp.float32)
        buf[slot] = jnp.where((q > 0.5)[None], m, imgf)
        pltpu.async_copy(buf.at[slot], out_ref.at[b], sem.at[slot],
                         priority=j % 2)

    @pl.when(i == pl.num_programs(0) - 1)
    def _drain():
        for k in range(RING):
            b = nb - RING + k
            pltpu.make_async_copy(buf.at[b % RING], out_ref.at[b],
                                  sem.at[b % RING]).wait()


def _constants():
    sel = np.zeros((NT, GRID, 128), dtype=np.float32)
    psc = np.zeros((NT, FR, GRID), dtype=np.float32)
    for t in range(NT):
        for l in range(128):
            sel[t, ((128 * t + l) % IMG_W) // SPW, l] = 1.0
        for r in range(FR):
            if r % NT == t:
                psc[t, r, r // 28] = 1.0
    return jnp.asarray(psc), jnp.asarray(sel)


@jax.jit
def kernel(x, image):
    batch = x.shape[0]
    xg = x.reshape(batch, GRID, GRID).astype(jnp.float32)
    img_flat = image.reshape(CH, FR, 128)
    psc, sel = _constants()

    mean = pl.pallas_call(
        _mean_body,
        out_shape=jax.ShapeDtypeStruct((1, 1), jnp.float32),
        in_specs=[pl.BlockSpec((CH, FR, 128), lambda: (0, 0, 0))],
        out_specs=pl.BlockSpec(memory_space=pltpu.SMEM),
    )(img_flat)

    out = pl.pallas_call(
        _pix_body,
        grid=(batch // BBLK,),
        out_shape=jax.ShapeDtypeStruct((batch, CH, FR, 128), jnp.float32),
        in_specs=[
            pl.BlockSpec((BBLK, GRID, GRID), lambda i: (i, 0, 0)),
            pl.BlockSpec((CH, FR, 128), lambda i: (0, 0, 0)),
            pl.BlockSpec((NT, FR, GRID), lambda i: (0, 0, 0)),
            pl.BlockSpec((NT, GRID, 128), lambda i: (0, 0, 0)),
            pl.BlockSpec(memory_space=pltpu.SMEM),
        ],
        out_specs=pl.BlockSpec(memory_space=pl.ANY),
        scratch_shapes=[
            pltpu.VMEM((RING, CH, FR, 128), jnp.float32),
            pltpu.SemaphoreType.DMA((RING,)),
        ],
    )(xg, img_flat, psc, sel, mean)
    return out.reshape(batch, CH, IMG_W, IMG_W)


# speedup vs baseline: 1.8088x; 1.8088x over previous
import jax
import jax.numpy as jnp
import numpy as np
from jax.experimental import pallas as pl
from jax.experimental.pallas import tpu as pltpu

SPW = 16
IMG_W = 224
GRID = IMG_W // SPW
CH = 3
BBLK = 4
FR = 392
NT = 7
NP = 4
PR = FR // NP


def _mean_body(img_ref, out_ref):
    out_ref[0, 0] = jnp.sum(img_ref[...]) * (1.0 / (CH * IMG_W * IMG_W))


def _pix_body(xg_ref, imgf_ref, d_ref, psc_ref, sel_ref, out_ref, buf, sem):
    i = pl.program_id(0)
    nb = pl.num_programs(0) * BBLK

    ws = []
    for j in range(BBLK):
        g = xg_ref[j]
        parts = [
            jnp.dot(g, sel_ref[t],
                    preferred_element_type=jnp.float32).astype(jnp.bfloat16)
            for t in range(NT)
        ]
        ws.append(jnp.concatenate(parts, axis=0))
    wcat = jnp.concatenate(ws, axis=1)

    for j in range(BBLK):
        @pl.when(i > 0)
        def _wait_prev():
            pltpu.make_async_copy(buf.at[j], out_ref.at[i * BBLK + j - BBLK],
                                  sem.at[j]).wait()

    for p in range(NP):
        rows = pl.ds(PR * p, PR)
        qp = jnp.dot(psc_ref[rows, :], wcat,
                     preferred_element_type=jnp.float32)
        imgc = imgf_ref[:, rows]
        dc = d_ref[:, rows]
        for j in range(BBLK):
            qj = qp[:, 128 * j:128 * (j + 1)]
            buf[j, :, rows] = imgc + qj[None] * dc

    for j in range(BBLK):
        pltpu.async_copy(buf.at[j], out_ref.at[i * BBLK + j], sem.at[j],
                         priority=j % 2)

    @pl.when(i == pl.num_programs(0) - 1)
    def _drain():
        for j in range(BBLK):
            pltpu.make_async_copy(buf.at[j], out_ref.at[nb - BBLK + j],
                                  sem.at[j]).wait()


def _constants():
    sel = np.zeros((NT, GRID, 128), dtype=np.float32)
    psc = np.zeros((FR, NT * GRID), dtype=np.float32)
    for t in range(NT):
        for l in range(128):
            sel[t, ((128 * t + l) % IMG_W) // SPW, l] = 1.0
    for r in range(FR):
        psc[r, GRID * (r % NT) + r // 28] = 1.0
    return (jnp.asarray(psc).astype(jnp.bfloat16),
            jnp.asarray(sel).astype(jnp.bfloat16))


@jax.jit
def kernel(x, image):
    batch = x.shape[0]
    xg = x.reshape(batch, GRID, GRID).astype(jnp.bfloat16)
    img_flat = image.reshape(CH, FR, 128)
    psc, sel = _constants()

    mean = pl.pallas_call(
        _mean_body,
        out_shape=jax.ShapeDtypeStruct((1, 1), jnp.float32),
        in_specs=[pl.BlockSpec((CH, FR, 128), lambda: (0, 0, 0))],
        out_specs=pl.BlockSpec(memory_space=pltpu.SMEM),
    )(img_flat)
    d_flat = mean.reshape(1, 1, 1) - img_flat

    out = pl.pallas_call(
        _pix_body,
        grid=(batch // BBLK,),
        out_shape=jax.ShapeDtypeStruct((batch, CH, FR, 128), jnp.float32),
        in_specs=[
            pl.BlockSpec((BBLK, GRID, GRID), lambda i: (i, 0, 0)),
            pl.BlockSpec((CH, FR, 128), lambda i: (0, 0, 0)),
            pl.BlockSpec((CH, FR, 128), lambda i: (0, 0, 0)),
            pl.BlockSpec((FR, NT * GRID), lambda i: (0, 0)),
            pl.BlockSpec((NT, GRID, 128), lambda i: (0, 0, 0)),
        ],
        out_specs=pl.BlockSpec(memory_space=pl.ANY),
        scratch_shapes=[
            pltpu.VMEM((BBLK, CH, FR, 128), jnp.float32),
            pltpu.SemaphoreType.DMA((BBLK,)),
        ],
    )(xg, img_flat, d_flat, psc, sel)
    return out.reshape(batch, CH, IMG_W, IMG_W)


# speedup vs baseline: 1.9843x; 1.0970x over previous
import jax
import jax.numpy as jnp
import numpy as np
from jax.experimental import pallas as pl
from jax.experimental.pallas import tpu as pltpu

SPW = 16
IMG_W = 224
GRID = IMG_W // SPW
CH = 3
BBLK = 8
FR = 392
NT = 7
NP = 8
PR = FR // NP


def _mean_body(img_ref, out_ref):
    out_ref[0, 0] = jnp.sum(img_ref[...]) * (1.0 / (CH * IMG_W * IMG_W))


def _pix_body(xg_ref, imgf_ref, d_ref, psc_ref, sel_ref, out_ref, buf, sem):
    i = pl.program_id(0)
    nb = pl.num_programs(0) * BBLK

    ws = []
    for j in range(BBLK):
        g = xg_ref[j]
        parts = [
            jnp.dot(g, sel_ref[t],
                    preferred_element_type=jnp.float32).astype(jnp.bfloat16)
            for t in range(NT)
        ]
        ws.append(jnp.concatenate(parts, axis=0))
    wcat = jnp.concatenate(ws, axis=1)

    for j in range(BBLK):
        @pl.when(i > 0)
        def _wait_prev():
            pltpu.make_async_copy(buf.at[j], out_ref.at[i * BBLK + j - BBLK],
                                  sem.at[j]).wait()

    for p in range(NP):
        rows = pl.ds(PR * p, PR)
        qp = jnp.dot(psc_ref[rows, :], wcat,
                     preferred_element_type=jnp.float32)
        imgc = imgf_ref[:, rows]
        dc = d_ref[:, rows]
        for j in range(BBLK):
            qj = qp[:, 128 * j:128 * (j + 1)]
            buf[j, :, rows] = imgc + qj[None] * dc

    for j in range(BBLK):
        pltpu.async_copy(buf.at[j], out_ref.at[i * BBLK + j], sem.at[j],
                         priority=j % 2)

    @pl.when(i == pl.num_programs(0) - 1)
    def _drain():
        for j in range(BBLK):
            pltpu.make_async_copy(buf.at[j], out_ref.at[nb - BBLK + j],
                                  sem.at[j]).wait()


def _constants():
    sel = np.zeros((NT, GRID, 128), dtype=np.float32)
    psc = np.zeros((FR, NT * GRID), dtype=np.float32)
    for t in range(NT):
        for l in range(128):
            sel[t, ((128 * t + l) % IMG_W) // SPW, l] = 1.0
    for r in range(FR):
        psc[r, GRID * (r % NT) + r // 28] = 1.0
    return (jnp.asarray(psc).astype(jnp.bfloat16),
            jnp.asarray(sel).astype(jnp.bfloat16))


@jax.jit
def kernel(x, image):
    batch = x.shape[0]
    xg = x.reshape(batch, GRID, GRID).astype(jnp.bfloat16)
    img_flat = image.reshape(CH, FR, 128)
    psc, sel = _constants()

    mean = pl.pallas_call(
        _mean_body,
        out_shape=jax.ShapeDtypeStruct((1, 1), jnp.float32),
        in_specs=[pl.BlockSpec((CH, FR, 128), lambda: (0, 0, 0))],
        out_specs=pl.BlockSpec(memory_space=pltpu.SMEM),
    )(img_flat)
    d_flat = mean.reshape(1, 1, 1) - img_flat

    out = pl.pallas_call(
        _pix_body,
        grid=(batch // BBLK,),
        out_shape=jax.ShapeDtypeStruct((batch, CH, FR, 128), jnp.float32),
        in_specs=[
            pl.BlockSpec((BBLK, GRID, GRID), lambda i: (i, 0, 0)),
            pl.BlockSpec((CH, FR, 128), lambda i: (0, 0, 0)),
            pl.BlockSpec((CH, FR, 128), lambda i: (0, 0, 0)),
            pl.BlockSpec((FR, NT * GRID), lambda i: (0, 0)),
            pl.BlockSpec((NT, GRID, 128), lambda i: (0, 0, 0)),
        ],
        out_specs=pl.BlockSpec(memory_space=pl.ANY),
        scratch_shapes=[
            pltpu.VMEM((BBLK, CH, FR, 128), jnp.float32),
            pltpu.SemaphoreType.DMA((BBLK,)),
        ],
    )(xg, img_flat, d_flat, psc, sel)
    return out.reshape(batch, CH, IMG_W, IMG_W)


# speedup vs baseline: 1.9911x; 1.0034x over previous
import jax
import jax.numpy as jnp
import numpy as np
from jax.experimental import pallas as pl
from jax.experimental.pallas import tpu as pltpu

SPW = 16
IMG_W = 224
GRID = IMG_W // SPW
CH = 3
BBLK = 8
FR = 392
NT = 7
NP = 8
PR = FR // NP


def _mean_body(img_ref, out_ref):
    out_ref[0, 0] = jnp.sum(img_ref[...]) * (1.0 / (CH * IMG_W * IMG_W))


def _pix_body(xg_ref, imgf_ref, d_ref, psc_ref, sel_ref, out_ref, buf, sem):
    i = pl.program_id(0)
    nb = pl.num_programs(0) * BBLK

    ws = []
    for j in range(BBLK):
        g = xg_ref[j]
        parts = [
            jnp.dot(g, sel_ref[t],
                    preferred_element_type=jnp.float32).astype(jnp.bfloat16)
            for t in range(NT)
        ]
        ws.append(jnp.concatenate(parts, axis=0))
    wcat = jnp.concatenate(ws, axis=1)

    for j in range(BBLK):
        @pl.when(i > 0)
        def _wait_prev():
            pltpu.make_async_copy(buf.at[j], out_ref.at[i * BBLK + j - BBLK],
                                  sem.at[j]).wait()

    for p in range(NP):
        rows = pl.ds(PR * p, PR)
        qp = jnp.dot(psc_ref[rows, :], wcat,
                     preferred_element_type=jnp.float32)
        imgc = imgf_ref[:, rows]
        dc = d_ref[:, rows]
        for j in range(BBLK):
            qj = qp[:, 128 * j:128 * (j + 1)]
            buf[j, :, rows] = imgc + qj[None] * dc

    for j in range(BBLK):
        pltpu.make_async_copy(buf.at[j], out_ref.at[i * BBLK + j],
                              sem.at[j]).start()

    @pl.when(i == pl.num_programs(0) - 1)
    def _drain():
        for j in range(BBLK):
            pltpu.make_async_copy(buf.at[j], out_ref.at[nb - BBLK + j],
                                  sem.at[j]).wait()


def _constants():
    sel = np.zeros((NT, GRID, 128), dtype=np.float32)
    psc = np.zeros((FR, NT * GRID), dtype=np.float32)
    for t in range(NT):
        for l in range(128):
            sel[t, ((128 * t + l) % IMG_W) // SPW, l] = 1.0
    for r in range(FR):
        psc[r, GRID * (r % NT) + r // 28] = 1.0
    return (jnp.asarray(psc).astype(jnp.bfloat16),
            jnp.asarray(sel).astype(jnp.bfloat16))


@jax.jit
def kernel(x, image):
    batch = x.shape[0]
    xg = x.reshape(batch, GRID, GRID).astype(jnp.bfloat16)
    img_flat = image.reshape(CH, FR, 128)
    psc, sel = _constants()

    mean = pl.pallas_call(
        _mean_body,
        out_shape=jax.ShapeDtypeStruct((1, 1), jnp.float32),
        in_specs=[pl.BlockSpec((CH, FR, 128), lambda: (0, 0, 0))],
        out_specs=pl.BlockSpec(memory_space=pltpu.SMEM),
    )(img_flat)
    d_flat = mean.reshape(1, 1, 1) - img_flat

    out = pl.pallas_call(
        _pix_body,
        grid=(batch // BBLK,),
        out_shape=jax.ShapeDtypeStruct((batch, CH, FR, 128), jnp.float32),
        in_specs=[
            pl.BlockSpec((BBLK, GRID, GRID), lambda i: (i, 0, 0)),
            pl.BlockSpec((CH, FR, 128), lambda i: (0, 0, 0)),
            pl.BlockSpec((CH, FR, 128), lambda i: (0, 0, 0)),
            pl.BlockSpec((FR, NT * GRID), lambda i: (0, 0)),
            pl.BlockSpec((NT, GRID, 128), lambda i: (0, 0, 0)),
        ],
        out_specs=pl.BlockSpec(memory_space=pl.ANY),
        scratch_shapes=[
            pltpu.VMEM((BBLK, CH, FR, 128), jnp.float32),
            pltpu.SemaphoreType.DMA((BBLK,)),
        ],
    )(xg, img_flat, d_flat, psc, sel)
    return out.reshape(batch, CH, IMG_W, IMG_W)


# speedup vs baseline: 2.3761x; 1.1934x over previous
import jax
import jax.numpy as jnp
import numpy as np
from jax.experimental import pallas as pl
from jax.experimental.pallas import tpu as pltpu

SPW = 16
IMG_W = 224
GRID = IMG_W // SPW
CH = 3
BBLK = 8
FR = 392
NT = 7
NP = 8
PR = FR // NP


def _mean_body(img_ref, out_ref):
    out_ref[0, 0] = jnp.sum(img_ref[...]) * (1.0 / (CH * IMG_W * IMG_W))


def _pix_body(xg_ref, imgf_ref, d_ref, psc_ref, sel_ref, out_ref, buf, sem):
    i = pl.program_id(0)
    nb = pl.num_programs(0) * BBLK
    half = (i % 2) * BBLK

    ws = []
    for j in range(BBLK):
        g = xg_ref[j]
        parts = [
            jnp.dot(g, sel_ref[t],
                    preferred_element_type=jnp.float32).astype(jnp.bfloat16)
            for t in range(NT)
        ]
        ws.append(jnp.concatenate(parts, axis=0))
    wcat = jnp.concatenate(ws, axis=1)

    for j in range(BBLK):
        @pl.when(i > 1)
        def _wait_prev():
            pltpu.make_async_copy(buf.at[half + j],
                                  out_ref.at[(i - 2) * BBLK + j],
                                  sem.at[half + j]).wait()

    for p in range(NP):
        rows = pl.ds(PR * p, PR)
        qp = jnp.dot(psc_ref[rows, :], wcat,
                     preferred_element_type=jnp.float32)
        imgc = imgf_ref[:, rows]
        dc = d_ref[:, rows]
        for j in range(BBLK):
            qj = qp[:, 128 * j:128 * (j + 1)]
            buf[half + j, :, rows] = imgc + qj[None] * dc

    for j in range(BBLK):
        pltpu.make_async_copy(buf.at[half + j], out_ref.at[i * BBLK + j],
                              sem.at[half + j]).start()

    @pl.when(i >= pl.num_programs(0) - 2)
    def _drain():
        for j in range(BBLK):
            pltpu.make_async_copy(buf.at[half + j], out_ref.at[i * BBLK + j],
                                  sem.at[half + j]).wait()


def _constants():
    sel = np.zeros((NT, GRID, 128), dtype=np.float32)
    psc = np.zeros((FR, NT * GRID), dtype=np.float32)
    for t in range(NT):
        for l in range(128):
            sel[t, ((128 * t + l) % IMG_W) // SPW, l] = 1.0
    for r in range(FR):
        psc[r, GRID * (r % NT) + r // 28] = 1.0
    return (jnp.asarray(psc).astype(jnp.bfloat16),
            jnp.asarray(sel).astype(jnp.bfloat16))


@jax.jit
def kernel(x, image):
    batch = x.shape[0]
    xg = x.reshape(batch, GRID, GRID).astype(jnp.bfloat16)
    img_flat = image.reshape(CH, FR, 128)
    psc, sel = _constants()

    mean = pl.pallas_call(
        _mean_body,
        out_shape=jax.ShapeDtypeStruct((1, 1), jnp.float32),
        in_specs=[pl.BlockSpec((CH, FR, 128), lambda: (0, 0, 0))],
        out_specs=pl.BlockSpec(memory_space=pltpu.SMEM),
    )(img_flat)
    d_flat = mean.reshape(1, 1, 1) - img_flat

    out = pl.pallas_call(
        _pix_body,
        grid=(batch // BBLK,),
        out_shape=jax.ShapeDtypeStruct((batch, CH, FR, 128), jnp.float32),
        in_specs=[
            pl.BlockSpec((BBLK, GRID, GRID), lambda i: (i, 0, 0)),
            pl.BlockSpec((CH, FR, 128), lambda i: (0, 0, 0)),
            pl.BlockSpec((CH, FR, 128), lambda i: (0, 0, 0)),
            pl.BlockSpec((FR, NT * GRID), lambda i: (0, 0)),
            pl.BlockSpec((NT, GRID, 128), lambda i: (0, 0, 0)),
        ],
        out_specs=pl.BlockSpec(memory_space=pl.ANY),
        scratch_shapes=[
            pltpu.VMEM((2 * BBLK, CH, FR, 128), jnp.float32),
            pltpu.SemaphoreType.DMA((2 * BBLK,)),
        ],
    )(xg, img_flat, d_flat, psc, sel)
    return out.reshape(batch, CH, IMG_W, IMG_W)


# speedup vs baseline: 2.4006x; 1.0103x over previous
import jax
import jax.numpy as jnp
import numpy as np
from jax.experimental import pallas as pl
from jax.experimental.pallas import tpu as pltpu

SPW = 16
IMG_W = 224
GRID = IMG_W // SPW
CH = 3
BBLK = 8
FR = 392
NT = 7
NP = 4
PR = FR // NP


def _mean_body(img_ref, out_ref):
    out_ref[0, 0] = jnp.sum(img_ref[...]) * (1.0 / (CH * IMG_W * IMG_W))


def _pix_body(xg_ref, imgf_ref, d_ref, psc_ref, sel_ref, out_ref, buf, sem):
    i = pl.program_id(0)
    nb = pl.num_programs(0) * BBLK
    half = (i % 2) * BBLK

    ws = []
    for j in range(BBLK):
        g = xg_ref[j]
        parts = [
            jnp.dot(g, sel_ref[t],
                    preferred_element_type=jnp.float32).astype(jnp.bfloat16)
            for t in range(NT)
        ]
        ws.append(jnp.concatenate(parts, axis=0))
    wcat = jnp.concatenate(ws, axis=1)

    for j in range(BBLK):
        @pl.when(i > 1)
        def _wait_prev():
            pltpu.make_async_copy(buf.at[half + j],
                                  out_ref.at[(i - 2) * BBLK + j],
                                  sem.at[half + j]).wait()

    for p in range(NP):
        rows = pl.ds(PR * p, PR)
        qp = jnp.dot(psc_ref[rows, :], wcat,
                     preferred_element_type=jnp.float32)
        imgc = imgf_ref[:, rows]
        dc = d_ref[:, rows]
        for j in range(BBLK):
            qj = qp[:, 128 * j:128 * (j + 1)]
            buf[half + j, :, rows] = imgc + qj[None] * dc

    for j in range(BBLK):
        pltpu.make_async_copy(buf.at[half + j], out_ref.at[i * BBLK + j],
                              sem.at[half + j]).start()

    @pl.when(i >= pl.num_programs(0) - 2)
    def _drain():
        for j in range(BBLK):
            pltpu.make_async_copy(buf.at[half + j], out_ref.at[i * BBLK + j],
                                  sem.at[half + j]).wait()


def _constants():
    sel = np.zeros((NT, GRID, 128), dtype=np.float32)
    psc = np.zeros((FR, NT * GRID), dtype=np.float32)
    for t in range(NT):
        for l in range(128):
            sel[t, ((128 * t + l) % IMG_W) // SPW, l] = 1.0
    for r in range(FR):
        psc[r, GRID * (r % NT) + r // 28] = 1.0
    return (jnp.asarray(psc).astype(jnp.bfloat16),
            jnp.asarray(sel).astype(jnp.bfloat16))


@jax.jit
def kernel(x, image):
    batch = x.shape[0]
    xg = x.reshape(batch, GRID, GRID).astype(jnp.bfloat16)
    img_flat = image.reshape(CH, FR, 128)
    psc, sel = _constants()

    mean = pl.pallas_call(
        _mean_body,
        out_shape=jax.ShapeDtypeStruct((1, 1), jnp.float32),
        in_specs=[pl.BlockSpec((CH, FR, 128), lambda: (0, 0, 0))],
        out_specs=pl.BlockSpec(memory_space=pltpu.SMEM),
    )(img_flat)
    d_flat = mean.reshape(1, 1, 1) - img_flat

    out = pl.pallas_call(
        _pix_body,
        grid=(batch // BBLK,),
        out_shape=jax.ShapeDtypeStruct((batch, CH, FR, 128), jnp.float32),
        in_specs=[
            pl.BlockSpec((BBLK, GRID, GRID), lambda i: (i, 0, 0)),
            pl.BlockSpec((CH, FR, 128), lambda i: (0, 0, 0)),
            pl.BlockSpec((CH, FR, 128), lambda i: (0, 0, 0)),
            pl.BlockSpec((FR, NT * GRID), lambda i: (0, 0)),
            pl.BlockSpec((NT, GRID, 128), lambda i: (0, 0, 0)),
        ],
        out_specs=pl.BlockSpec(memory_space=pl.ANY),
        scratch_shapes=[
            pltpu.VMEM((2 * BBLK, CH, FR, 128), jnp.float32),
            pltpu.SemaphoreType.DMA((2 * BBLK,)),
        ],
    )(xg, img_flat, d_flat, psc, sel)
    return out.reshape(batch, CH, IMG_W, IMG_W)
